# Initial kernel scaffold; baseline (speedup 1.0000x reference)
import jax
import jax.numpy as jnp
from jax.experimental import pallas as pl


def kernel(properties, z):
    raise NotImplementedError("write your pallas kernel here")


# speedup vs baseline: 3.7636x; 3.7636x over previous
import functools

import jax
import jax.numpy as jnp
from jax import lax
from jax.experimental import pallas as pl
from jax.experimental.pallas import tpu as pltpu
from jax.experimental.pallas import tpu_sc as plsc

_NUM_WORKERS = 32
_CHUNK = 128


def _build_gather(num_rows, d, b):
    b_per_w = b // _NUM_WORKERS
    n_chunks = b_per_w // _CHUNK
    mesh = plsc.VectorSubcoreMesh(core_axis_name="c", subcore_axis_name="s")

    @functools.partial(
        pl.kernel,
        mesh=mesh,
        out_type=jax.ShapeDtypeStruct((b, d), jnp.float32),
        scratch_types=[
            pltpu.VMEM((_CHUNK,), jnp.int32),
            pltpu.VMEM((_CHUNK, d), jnp.float32),
            pltpu.SemaphoreType.DMA,
        ],
        compiler_params=pltpu.CompilerParams(use_tc_tiling_on_sc=False),
    )
    def gather_kernel(table_hbm, idx_hbm, out_hbm, idx_v, rows_v, sem):
        wid = lax.axis_index("s") * 2 + lax.axis_index("c")
        base = wid * b_per_w

        def body(i, carry):
            off = base + i * _CHUNK
            pltpu.sync_copy(idx_hbm.at[pl.ds(off, _CHUNK)], idx_v)
            pltpu.async_copy(table_hbm.at[idx_v], rows_v, sem).wait()
            pltpu.sync_copy(rows_v, out_hbm.at[pl.ds(off, _CHUNK)])
            return carry

        lax.fori_loop(0, n_chunks, body, 0)

    return gather_kernel


def kernel(properties, z):
    num_rows, d = properties.shape
    zf = z.reshape(-1).astype(jnp.int32)
    b = zf.shape[0]
    out = _build_gather(num_rows, d, b)(properties, zf)
    return out.reshape(z.shape + (d,))


# speedup vs baseline: 5.1591x; 1.3708x over previous
import functools

import jax
import jax.numpy as jnp
from jax import lax
from jax.experimental import pallas as pl
from jax.experimental.pallas import tpu as pltpu
from jax.experimental.pallas import tpu_sc as plsc

_NUM_WORKERS = 32
_CH = 128
_K = 4
_SUP = _K * _CH


def _build_gather(num_rows, d, b):
    b_per_w = b // _NUM_WORKERS
    n_sup = b_per_w // _SUP
    sup_per_w_pairs = n_sup // 2
    mesh = plsc.VectorSubcoreMesh(core_axis_name="c", subcore_axis_name="s")

    @functools.partial(
        pl.kernel,
        mesh=mesh,
        out_type=jax.ShapeDtypeStruct((b, d), jnp.float32),
        scratch_types=[
            pltpu.VMEM((_K, _CH), jnp.int32),
            pltpu.VMEM((_K, _CH), jnp.int32),
            pltpu.VMEM((_SUP, d), jnp.float32),
            pltpu.VMEM((_SUP, d), jnp.float32),
            pltpu.SemaphoreType.DMA,
            pltpu.SemaphoreType.DMA,
            pltpu.SemaphoreType.DMA,
        ],
        compiler_params=pltpu.CompilerParams(use_tc_tiling_on_sc=False),
    )
    def gather_kernel(table_hbm, idx_hbm, out_hbm, ibuf0, ibuf1, rbuf0, rbuf1,
                      idx_sem, gat_sem, out_sem):
        wid = lax.axis_index("s") * 2 + lax.axis_index("c")
        row0 = wid * (b_per_w // _CH)
        base = wid * b_per_w

        pltpu.async_copy(idx_hbm.at[pl.ds(row0, _K)], ibuf0, idx_sem)
        pltpu.async_copy(idx_hbm.at[pl.ds(row0 + _K, _K)], ibuf1, idx_sem)

        def do_superchunk(sc, ibuf, rbuf):
            out_slice = out_hbm.at[pl.ds(base + sc * _SUP, _SUP)]
            @pl.when(sc >= 2)
            def _():
                pltpu.make_async_copy(rbuf, out_slice, out_sem).wait()

            pltpu.make_async_copy(idx_hbm.at[pl.ds(row0, _K)], ibuf,
                                  idx_sem).wait()
            for j in range(_K):
                pltpu.async_copy(table_hbm.at[ibuf.at[j]],
                                 rbuf.at[pl.ds(j * _CH, _CH)], gat_sem)
            for j in range(_K):
                pltpu.make_async_copy(table_hbm.at[ibuf.at[j]],
                                      rbuf.at[pl.ds(j * _CH, _CH)],
                                      gat_sem).wait()
            @pl.when(sc + 2 < n_sup)
            def _():
                pltpu.async_copy(
                    idx_hbm.at[pl.ds(row0 + (sc + 2) * _K, _K)], ibuf, idx_sem)

            pltpu.async_copy(rbuf, out_slice, out_sem)

        def body(t, carry):
            do_superchunk(2 * t, ibuf0, rbuf0)
            do_superchunk(2 * t + 1, ibuf1, rbuf1)
            return carry

        lax.fori_loop(0, sup_per_w_pairs, body, 0)

        tail = out_hbm.at[pl.ds(base, _SUP)]
        pltpu.make_async_copy(rbuf0, tail, out_sem).wait()
        pltpu.make_async_copy(rbuf1, tail, out_sem).wait()

    return gather_kernel


def kernel(properties, z):
    num_rows, d = properties.shape
    zf = z.reshape(-1).astype(jnp.int32)
    b = zf.shape[0]
    idx2d = zf.reshape(b // _CH, _CH)
    out = _build_gather(num_rows, d, b)(properties, idx2d)
    return out.reshape(z.shape + (d,))


# speedup vs baseline: 5.1717x; 1.0024x over previous
import functools

import jax
import jax.numpy as jnp
from jax import lax
from jax.experimental import pallas as pl
from jax.experimental.pallas import tpu as pltpu
from jax.experimental.pallas import tpu_sc as plsc

_NUM_WORKERS = 32
_CH = 128
_K = 5
_SUP = _K * _CH


def _build_gather(num_rows, d, b):
    b_per_w = b // _NUM_WORKERS
    n_sup = b_per_w // _SUP
    idx_rows_per_w = b_per_w // _CH
    mesh = plsc.VectorSubcoreMesh(core_axis_name="c", subcore_axis_name="s")

    @functools.partial(
        pl.kernel,
        mesh=mesh,
        out_type=jax.ShapeDtypeStruct((b, d), jnp.float32),
        scratch_types=[
            pltpu.VMEM((_K, _CH), jnp.int32),
            pltpu.VMEM((_K, _CH), jnp.int32),
            pltpu.VMEM((_SUP, d), jnp.float32),
            pltpu.VMEM((_SUP, d), jnp.float32),
            pltpu.SemaphoreType.DMA,
            pltpu.SemaphoreType.DMA,
            pltpu.SemaphoreType.DMA,
            pltpu.SemaphoreType.DMA,
            pltpu.SemaphoreType.DMA,
        ],
        compiler_params=pltpu.CompilerParams(use_tc_tiling_on_sc=False),
    )
    def gather_kernel(table_hbm, idx_hbm, out_hbm, ibuf0, ibuf1, rbuf0, rbuf1,
                      isem0, isem1, gsem0, gsem1, out_sem):
        wid = lax.axis_index("s") * 2 + lax.axis_index("c")
        row0 = wid * idx_rows_per_w
        base = wid * b_per_w

        def idx_copy(sc, ibuf, isem):
            return pltpu.make_async_copy(
                idx_hbm.at[pl.ds(row0 + sc * _K, _K)], ibuf, isem)

        def gather_copies(ibuf, rbuf, gsem):
            return [
                pltpu.make_async_copy(table_hbm.at[ibuf.at[j]],
                                      rbuf.at[pl.ds(j * _CH, _CH)], gsem)
                for j in range(_K)
            ]

        def out_copy(sc, rbuf):
            return pltpu.make_async_copy(
                rbuf, out_hbm.at[pl.ds(base + sc * _SUP, _SUP)], out_sem)

        idx_copy(0, ibuf0, isem0).start()
        idx_copy(1, ibuf1, isem1).start()
        idx_copy(0, ibuf0, isem0).wait()
        for c in gather_copies(ibuf0, rbuf0, gsem0):
            c.start()
        idx_copy(1, ibuf1, isem1).wait()
        for c in gather_copies(ibuf1, rbuf1, gsem1):
            c.start()
        for c in gather_copies(ibuf0, rbuf0, gsem0):
            c.wait()
        out_copy(0, rbuf0).start()
        idx_copy(2, ibuf0, isem0).start()

        def half(sc, ibuf_cur, rbuf_cur, gsem_cur, isem_cur, ibuf_prv,
                 rbuf_prv, gsem_prv, isem_prv):
            out_copy(sc - 2, rbuf_cur).wait()
            idx_copy(sc, ibuf_cur, isem_cur).wait()
            for c in gather_copies(ibuf_cur, rbuf_cur, gsem_cur):
                c.start()
            for c in gather_copies(ibuf_prv, rbuf_prv, gsem_prv):
                c.wait()
            out_copy(sc - 1, rbuf_prv).start()

            @pl.when(sc + 1 < n_sup)
            def _():
                idx_copy(sc + 1, ibuf_prv, isem_prv).start()

        def body(t, carry):
            sc = 2 * t + 2
            half(sc, ibuf0, rbuf0, gsem0, isem0, ibuf1, rbuf1, gsem1, isem1)
            half(sc + 1, ibuf1, rbuf1, gsem1, isem1, ibuf0, rbuf0, gsem0,
                 isem0)
            return carry

        lax.fori_loop(0, (n_sup - 2) // 2, body, 0)

        out_copy(n_sup - 2, rbuf0).wait()
        for c in gather_copies(ibuf1, rbuf1, gsem1):
            c.wait()
        out_copy(n_sup - 1, rbuf1).start()
        out_copy(n_sup - 1, rbuf1).wait()

    return gather_kernel


def kernel(properties, z):
    num_rows, d = properties.shape
    zf = z.reshape(-1).astype(jnp.int32)
    b = zf.shape[0]
    idx2d = zf.reshape(b // _CH, _CH)
    out = _build_gather(num_rows, d, b)(properties, idx2d)
    return out.reshape(z.shape + (d,))
